# baseline (speedup 1.0000x reference)
import functools

import jax
import jax.numpy as jnp
from jax import lax
from jax.experimental import pallas as pl
from jax.experimental.pallas import tpu as pltpu
from jax.experimental.pallas import tpu_sc as plsc

B = 16384
BERT_DIM = 768
NEWS_DIM = 256
CAT_DIM = 128
QDIM = 200
QPAD = 256

NC = 2
NS = 16
NW = NC * NS
CHUNK = 64
SPLITS = (2048,) * 8


@functools.cache
def _make_sc_gather(rows):
    BPW = rows // NW
    NCH = BPW // CHUNK

    @functools.partial(
        pl.kernel,
        mesh=plsc.VectorSubcoreMesh(core_axis_name="c", subcore_axis_name="s"),
        out_type=[
            jax.ShapeDtypeStruct((rows, BERT_DIM), jnp.float32),
            jax.ShapeDtypeStruct((rows, CAT_DIM), jnp.float32),
            jax.ShapeDtypeStruct((rows, CAT_DIM), jnp.float32),
        ],
        scratch_types=[
            pltpu.VMEM((BPW,), jnp.int32),
            pltpu.VMEM((BPW,), jnp.int32),
            pltpu.VMEM((BPW,), jnp.int32),
            pltpu.VMEM((CHUNK, BERT_DIM), jnp.float32),
            pltpu.VMEM((CHUNK, BERT_DIM), jnp.float32),
            pltpu.VMEM((CHUNK, CAT_DIM), jnp.float32),
            pltpu.VMEM((CHUNK, CAT_DIM), jnp.float32),
            pltpu.SemaphoreType.DMA,
            pltpu.SemaphoreType.DMA,
            pltpu.SemaphoreType.DMA,
        ],
    )
    def _sc_gather(idx0, idx1, idx2, news_t, cat_t, auth_t,
                   out_news, out_cat, out_auth,
                   i0_v, i1_v, i2_v, nb0, nb1, sb0, sb1, sem0, sem1, semi):
        wid = lax.axis_index("s") * NC + lax.axis_index("c")
        base = wid * BPW

        hi = [pltpu.async_copy(src.at[pl.ds(base, BPW)], dst, semi)
              for src, dst in ((idx0, i0_v), (idx1, i1_v), (idx2, i2_v))]
        for h in hi:
            h.wait()

        nbufs = (nb0, nb1)
        sbufs = (sb0, sb1)
        tasks = []
        nn = ns = 0
        for table, idx_v, out, bufs in ((news_t, i0_v, out_news, nbufs),
                                        (cat_t, i1_v, out_cat, sbufs),
                                        (auth_t, i2_v, out_auth, sbufs)):
            for j in range(NCH):
                if bufs is nbufs:
                    buf = bufs[nn % 2]
                    nn += 1
                else:
                    buf = bufs[ns % 2]
                    ns += 1
                tasks.append((table, idx_v, out, buf, j))

        sems = (sem0, sem1)
        n = len(tasks)
        pend = [None] * n
        pend_w = [None] * n

        def fire(k):
            table, idx_v, _, buf, j = tasks[k]
            pend[k] = pltpu.async_copy(
                table.at[idx_v.at[pl.ds(j * CHUNK, CHUNK)]], buf, sems[k % 2])

        fire(0)
        fire(1)
        for k in range(n):
            pend[k].wait()
            _, _, out, buf, j = tasks[k]
            pend_w[k] = pltpu.async_copy(
                buf, out.at[pl.ds(base + j * CHUNK, CHUNK)], semi)
            if k + 2 < n:
                pend_w[k].wait()
                fire(k + 2)
        pend_w[n - 2].wait()
        pend_w[n - 1].wait()

    return _sc_gather


BLK = 1024


def _dense_body(news_ref, cat_ref, auth_ref, wp_ref, bp_ref, g_ref, bb_ref,
                wc_ref, bc_ref, wa_ref, ba_ref, w1_ref, b1_ref, w2_ref,
                b2_ref, out_ref):
    bf = jnp.bfloat16
    h = jnp.dot(news_ref[...].astype(bf), wp_ref[...].astype(bf),
                preferred_element_type=jnp.float32) + bp_ref[...]
    mu = jnp.mean(h, axis=-1, keepdims=True)
    var = jnp.mean((h - mu) ** 2, axis=-1, keepdims=True)
    hn = (h - mu) * lax.rsqrt(var + 1e-5) * g_ref[...] + bb_ref[...]
    t = hn * jax.nn.sigmoid(hn)
    c = jnp.dot(cat_ref[...].astype(bf), wc_ref[...].astype(bf),
                preferred_element_type=jnp.float32) + bc_ref[...]
    a = jnp.dot(auth_ref[...].astype(bf), wa_ref[...].astype(bf),
                preferred_element_type=jnp.float32) + ba_ref[...]

    w1 = w1_ref[...].astype(bf)
    b1 = b1_ref[...]
    w2 = w2_ref[...]
    ab2 = b2_ref[0, 0]

    def score(v):
        e = jnp.tanh(jnp.dot(v.astype(bf), w1,
                             preferred_element_type=jnp.float32) + b1)
        return jnp.sum(e * w2, axis=-1, keepdims=True) + ab2

    at = jnp.exp(score(t))
    ac = jnp.exp(score(c))
    aa = jnp.exp(score(a))
    denom = at + ac + aa + 1e-8
    out_ref[...] = (t * at + c * ac + a * aa) / denom


def _dense(news_rows, cat_rows, auth_rows, wp, bp, g, bb, wc, bc, wa, ba,
           w1, b1, w2, b2, out_buf, blk_off):
    rows = news_rows.shape[0]
    grid = (rows // BLK,)
    row_spec = lambda d: pl.BlockSpec((BLK, d), lambda i: (i, 0))
    full = lambda s: pl.BlockSpec(s, lambda i: (0, 0))

    def body(*refs):
        _dense_body(*refs[:15], refs[-1])

    in_specs = [
        row_spec(BERT_DIM), row_spec(CAT_DIM), row_spec(CAT_DIM),
        full((BERT_DIM, NEWS_DIM)), full((1, NEWS_DIM)),
        full((1, NEWS_DIM)), full((1, NEWS_DIM)),
        full((CAT_DIM, NEWS_DIM)), full((1, NEWS_DIM)),
        full((CAT_DIM, NEWS_DIM)), full((1, NEWS_DIM)),
        full((NEWS_DIM, QPAD)), full((1, QPAD)), full((1, QPAD)),
        full((1, 1)),
    ]
    args = [news_rows, cat_rows, auth_rows, wp, bp, g, bb, wc, bc, wa, ba,
            w1, b1, w2, b2]
    aliases = {}
    if out_buf is not None:
        in_specs.append(pl.BlockSpec(memory_space=pl.ANY))
        args.append(out_buf)
        aliases = {15: 0}
    return pl.pallas_call(
        body,
        grid=grid,
        in_specs=in_specs,
        out_specs=pl.BlockSpec((BLK, NEWS_DIM), lambda i: (i + blk_off, 0)),
        out_shape=jax.ShapeDtypeStruct((B, NEWS_DIM), jnp.float32),
        input_output_aliases=aliases,
    )(*args)


def kernel(x, news_table, W_pool, b_pool, ln_g, ln_b, cat_table, W_cat,
           b_cat, auth_table, W_auth, b_auth, aW1, ab1, aW2, ab2):
    idx0 = x[:, 0]
    idx1 = x[:, 1]
    idx2 = x[:, 2]

    wp = W_pool.T
    wc = W_cat.T
    wa = W_auth.T
    w1 = jnp.zeros((NEWS_DIM, QPAD), jnp.float32).at[:, :QDIM].set(aW1.T)
    b1 = jnp.zeros((1, QPAD), jnp.float32).at[0, :QDIM].set(ab1)
    w2 = jnp.zeros((1, QPAD), jnp.float32).at[0, :QDIM].set(aW2[0])
    row = lambda v: v.reshape(1, -1)

    out = None
    lo = 0
    for rows in SPLITS:
        news_rows, cat_rows, auth_rows = _make_sc_gather(rows)(
            lax.dynamic_slice_in_dim(idx0, lo, rows),
            lax.dynamic_slice_in_dim(idx1, lo, rows),
            lax.dynamic_slice_in_dim(idx2, lo, rows),
            news_table, cat_table, auth_table)
        out = _dense(news_rows, cat_rows, auth_rows, wp, row(b_pool),
                     row(ln_g), row(ln_b), wc, row(b_cat), wa,
                     row(b_auth), w1, b1, w2, ab2.reshape(1, 1),
                     out, lo // BLK)
        lo += rows
    return out


# speedup vs baseline: 1.1280x; 1.1280x over previous
import functools

import jax
import jax.numpy as jnp
from jax import lax
from jax.experimental import pallas as pl
from jax.experimental.pallas import tpu as pltpu
from jax.experimental.pallas import tpu_sc as plsc

B = 16384
BERT_DIM = 768
NEWS_DIM = 256
CAT_DIM = 128
QDIM = 200
QPAD = 256

NC = 2
NS = 16
NW = NC * NS
CHUNK = 64
SPLITS = (4096, 4096, 4096, 4096)


@functools.cache
def _make_sc_gather(rows):
    BPW = rows // NW
    NCH = BPW // CHUNK

    @functools.partial(
        pl.kernel,
        mesh=plsc.VectorSubcoreMesh(core_axis_name="c", subcore_axis_name="s"),
        out_type=[
            jax.ShapeDtypeStruct((rows, BERT_DIM), jnp.float32),
            jax.ShapeDtypeStruct((rows, CAT_DIM), jnp.float32),
            jax.ShapeDtypeStruct((rows, CAT_DIM), jnp.float32),
        ],
        scratch_types=[
            pltpu.VMEM((BPW,), jnp.int32),
            pltpu.VMEM((BPW,), jnp.int32),
            pltpu.VMEM((BPW,), jnp.int32),
            pltpu.VMEM((CHUNK, BERT_DIM), jnp.float32),
            pltpu.VMEM((CHUNK, BERT_DIM), jnp.float32),
            pltpu.VMEM((CHUNK, CAT_DIM), jnp.float32),
            pltpu.VMEM((CHUNK, CAT_DIM), jnp.float32),
            pltpu.SemaphoreType.DMA,
            pltpu.SemaphoreType.DMA,
            pltpu.SemaphoreType.DMA,
        ],
    )
    def _sc_gather(idx0, idx1, idx2, news_t, cat_t, auth_t,
                   out_news, out_cat, out_auth,
                   i0_v, i1_v, i2_v, nb0, nb1, sb0, sb1, sem0, sem1, semi):
        wid = lax.axis_index("s") * NC + lax.axis_index("c")
        base = wid * BPW

        hi = [pltpu.async_copy(src.at[pl.ds(base, BPW)], dst, semi)
              for src, dst in ((idx0, i0_v), (idx1, i1_v), (idx2, i2_v))]
        for h in hi:
            h.wait()

        nbufs = (nb0, nb1)
        sbufs = (sb0, sb1)
        tasks = []
        nn = ns = 0
        for table, idx_v, out, bufs in ((news_t, i0_v, out_news, nbufs),
                                        (cat_t, i1_v, out_cat, sbufs),
                                        (auth_t, i2_v, out_auth, sbufs)):
            for j in range(NCH):
                if bufs is nbufs:
                    buf = bufs[nn % 2]
                    nn += 1
                else:
                    buf = bufs[ns % 2]
                    ns += 1
                tasks.append((table, idx_v, out, buf, j))

        sems = (sem0, sem1)
        n = len(tasks)
        pend = [None] * n
        pend_w = [None] * n

        def fire(k):
            table, idx_v, _, buf, j = tasks[k]
            pend[k] = pltpu.async_copy(
                table.at[idx_v.at[pl.ds(j * CHUNK, CHUNK)]], buf, sems[k % 2])

        fire(0)
        fire(1)
        for k in range(n):
            pend[k].wait()
            _, _, out, buf, j = tasks[k]
            pend_w[k] = pltpu.async_copy(
                buf, out.at[pl.ds(base + j * CHUNK, CHUNK)], semi)
            if k + 2 < n:
                pend_w[k].wait()
                fire(k + 2)
        pend_w[n - 2].wait()
        pend_w[n - 1].wait()

    return _sc_gather


BLK = 1024


def _dense_body(news_ref, cat_ref, auth_ref, wp_ref, bp_ref, g_ref, bb_ref,
                wc_ref, bc_ref, wa_ref, ba_ref, w1_ref, b1_ref, w2_ref,
                b2_ref, out_ref):
    bf = jnp.bfloat16
    h = jnp.dot(news_ref[...].astype(bf), wp_ref[...].astype(bf),
                preferred_element_type=jnp.float32) + bp_ref[...]
    mu = jnp.mean(h, axis=-1, keepdims=True)
    var = jnp.mean((h - mu) ** 2, axis=-1, keepdims=True)
    hn = (h - mu) * lax.rsqrt(var + 1e-5) * g_ref[...] + bb_ref[...]
    t = hn * jax.nn.sigmoid(hn)
    c = jnp.dot(cat_ref[...].astype(bf), wc_ref[...].astype(bf),
                preferred_element_type=jnp.float32) + bc_ref[...]
    a = jnp.dot(auth_ref[...].astype(bf), wa_ref[...].astype(bf),
                preferred_element_type=jnp.float32) + ba_ref[...]

    w1 = w1_ref[...].astype(bf)
    b1 = b1_ref[...]
    w2 = w2_ref[...]
    ab2 = b2_ref[0, 0]

    def score(v):
        e = jnp.tanh(jnp.dot(v.astype(bf), w1,
                             preferred_element_type=jnp.float32) + b1)
        return jnp.sum(e * w2, axis=-1, keepdims=True) + ab2

    at = jnp.exp(score(t))
    ac = jnp.exp(score(c))
    aa = jnp.exp(score(a))
    denom = at + ac + aa + 1e-8
    out_ref[...] = (t * at + c * ac + a * aa) / denom


def _dense(news_rows, cat_rows, auth_rows, wp, bp, g, bb, wc, bc, wa, ba,
           w1, b1, w2, b2, out_buf, blk_off):
    rows = news_rows.shape[0]
    grid = (rows // BLK,)
    row_spec = lambda d: pl.BlockSpec((BLK, d), lambda i: (i, 0))
    full = lambda s: pl.BlockSpec(s, lambda i: (0, 0))

    def body(*refs):
        _dense_body(*refs[:15], refs[-1])

    in_specs = [
        row_spec(BERT_DIM), row_spec(CAT_DIM), row_spec(CAT_DIM),
        full((BERT_DIM, NEWS_DIM)), full((1, NEWS_DIM)),
        full((1, NEWS_DIM)), full((1, NEWS_DIM)),
        full((CAT_DIM, NEWS_DIM)), full((1, NEWS_DIM)),
        full((CAT_DIM, NEWS_DIM)), full((1, NEWS_DIM)),
        full((NEWS_DIM, QPAD)), full((1, QPAD)), full((1, QPAD)),
        full((1, 1)),
    ]
    args = [news_rows, cat_rows, auth_rows, wp, bp, g, bb, wc, bc, wa, ba,
            w1, b1, w2, b2]
    aliases = {}
    if out_buf is not None:
        in_specs.append(pl.BlockSpec(memory_space=pl.ANY))
        args.append(out_buf)
        aliases = {15: 0}
    return pl.pallas_call(
        body,
        grid=grid,
        in_specs=in_specs,
        out_specs=pl.BlockSpec((BLK, NEWS_DIM), lambda i: (i + blk_off, 0)),
        out_shape=jax.ShapeDtypeStruct((B, NEWS_DIM), jnp.float32),
        input_output_aliases=aliases,
    )(*args)


def kernel(x, news_table, W_pool, b_pool, ln_g, ln_b, cat_table, W_cat,
           b_cat, auth_table, W_auth, b_auth, aW1, ab1, aW2, ab2):
    idx0 = x[:, 0]
    idx1 = x[:, 1]
    idx2 = x[:, 2]

    wp = W_pool.T
    wc = W_cat.T
    wa = W_auth.T
    w1 = jnp.zeros((NEWS_DIM, QPAD), jnp.float32).at[:, :QDIM].set(aW1.T)
    b1 = jnp.zeros((1, QPAD), jnp.float32).at[0, :QDIM].set(ab1)
    w2 = jnp.zeros((1, QPAD), jnp.float32).at[0, :QDIM].set(aW2[0])
    row = lambda v: v.reshape(1, -1)

    out = None
    lo = 0
    for rows in SPLITS:
        news_rows, cat_rows, auth_rows = _make_sc_gather(rows)(
            lax.dynamic_slice_in_dim(idx0, lo, rows),
            lax.dynamic_slice_in_dim(idx1, lo, rows),
            lax.dynamic_slice_in_dim(idx2, lo, rows),
            news_table, cat_table, auth_table)
        out = _dense(news_rows, cat_rows, auth_rows, wp, row(b_pool),
                     row(ln_g), row(ln_b), wc, row(b_cat), wa,
                     row(b_auth), w1, b1, w2, ab2.reshape(1, 1),
                     out, lo // BLK)
        lo += rows
    return out


# speedup vs baseline: 1.1316x; 1.0032x over previous
import functools

import jax
import jax.numpy as jnp
from jax import lax
from jax.experimental import pallas as pl
from jax.experimental.pallas import tpu as pltpu
from jax.experimental.pallas import tpu_sc as plsc

B = 16384
BERT_DIM = 768
NEWS_DIM = 256
CAT_DIM = 128
QDIM = 200
QPAD = 256

NC = 2
NS = 16
NW = NC * NS
CHUNK = 64
SPLITS = (4096, 4096, 4096, 4096)


@functools.cache
def _make_sc_gather(rows):
    BPW = rows // NW
    NCH = BPW // CHUNK

    @functools.partial(
        pl.kernel,
        mesh=plsc.VectorSubcoreMesh(core_axis_name="c", subcore_axis_name="s"),
        out_type=[
            jax.ShapeDtypeStruct((rows, BERT_DIM), jnp.float32),
            jax.ShapeDtypeStruct((rows, CAT_DIM), jnp.float32),
            jax.ShapeDtypeStruct((rows, CAT_DIM), jnp.float32),
        ],
        scratch_types=[
            pltpu.VMEM((BPW,), jnp.int32),
            pltpu.VMEM((BPW,), jnp.int32),
            pltpu.VMEM((BPW,), jnp.int32),
            pltpu.VMEM((CHUNK, BERT_DIM), jnp.float32),
            pltpu.VMEM((CHUNK, BERT_DIM), jnp.float32),
            pltpu.VMEM((CHUNK, CAT_DIM), jnp.float32),
            pltpu.VMEM((CHUNK, CAT_DIM), jnp.float32),
            pltpu.SemaphoreType.DMA,
            pltpu.SemaphoreType.DMA,
            pltpu.SemaphoreType.DMA,
        ],
    )
    def _sc_gather(idx0, idx1, idx2, news_t, cat_t, auth_t,
                   out_news, out_cat, out_auth,
                   i0_v, i1_v, i2_v, nb0, nb1, sb0, sb1, sem0, sem1, semi):
        wid = lax.axis_index("s") * NC + lax.axis_index("c")
        base = wid * BPW

        hi = [pltpu.async_copy(src.at[pl.ds(base, BPW)], dst, semi)
              for src, dst in ((idx0, i0_v), (idx1, i1_v), (idx2, i2_v))]
        for h in hi:
            h.wait()

        nbufs = (nb0, nb1)
        sbufs = (sb0, sb1)
        tasks = []
        nn = ns = 0
        for table, idx_v, out, bufs in ((news_t, i0_v, out_news, nbufs),
                                        (cat_t, i1_v, out_cat, sbufs),
                                        (auth_t, i2_v, out_auth, sbufs)):
            for j in range(NCH):
                if bufs is nbufs:
                    buf = bufs[nn % 2]
                    nn += 1
                else:
                    buf = bufs[ns % 2]
                    ns += 1
                tasks.append((table, idx_v, out, buf, j))

        sems = (sem0, sem1)
        n = len(tasks)
        pend = [None] * n
        pend_w = [None] * n

        def fire(k):
            table, idx_v, _, buf, j = tasks[k]
            pend[k] = pltpu.async_copy(
                table.at[idx_v.at[pl.ds(j * CHUNK, CHUNK)]], buf, sems[k % 2])

        fire(0)
        fire(1)
        for k in range(n):
            pend[k].wait()
            _, _, out, buf, j = tasks[k]
            pend_w[k] = pltpu.async_copy(
                buf, out.at[pl.ds(base + j * CHUNK, CHUNK)], semi)
            if k + 2 < n:
                pend_w[k].wait()
                fire(k + 2)
        pend_w[n - 2].wait()
        pend_w[n - 1].wait()

    return _sc_gather


BLK = 1024


def _dense_body(news_ref, cat_ref, auth_ref, wp_ref, bp_ref, g_ref, bb_ref,
                wc_ref, bc_ref, wa_ref, ba_ref, w1_ref, b1_ref, w2_ref,
                b2_ref, out_ref):
    h = jnp.dot(news_ref[...], wp_ref[...],
                preferred_element_type=jnp.float32) + bp_ref[...]
    mu = jnp.mean(h, axis=-1, keepdims=True)
    var = jnp.mean((h - mu) ** 2, axis=-1, keepdims=True)
    hn = (h - mu) * lax.rsqrt(var + 1e-5) * g_ref[...] + bb_ref[...]
    t = hn * jax.nn.sigmoid(hn)
    c = jnp.dot(cat_ref[...], wc_ref[...],
                preferred_element_type=jnp.float32) + bc_ref[...]
    a = jnp.dot(auth_ref[...], wa_ref[...],
                preferred_element_type=jnp.float32) + ba_ref[...]

    w1 = w1_ref[...]
    b1 = b1_ref[...]
    w2 = w2_ref[...]
    ab2 = b2_ref[0, 0]

    def score(v):
        e = jnp.tanh(jnp.dot(v, w1, preferred_element_type=jnp.float32) + b1)
        return jnp.sum(e * w2, axis=-1, keepdims=True) + ab2

    at = jnp.exp(score(t))
    ac = jnp.exp(score(c))
    aa = jnp.exp(score(a))
    denom = at + ac + aa + 1e-8
    out_ref[...] = (t * at + c * ac + a * aa) / denom


def _dense(news_rows, cat_rows, auth_rows, wp, bp, g, bb, wc, bc, wa, ba,
           w1, b1, w2, b2, out_buf, blk_off):
    rows = news_rows.shape[0]
    grid = (rows // BLK,)
    row_spec = lambda d: pl.BlockSpec((BLK, d), lambda i: (i, 0))
    full = lambda s: pl.BlockSpec(s, lambda i: (0, 0))

    def body(*refs):
        _dense_body(*refs[:15], refs[-1])

    in_specs = [
        row_spec(BERT_DIM), row_spec(CAT_DIM), row_spec(CAT_DIM),
        full((BERT_DIM, NEWS_DIM)), full((1, NEWS_DIM)),
        full((1, NEWS_DIM)), full((1, NEWS_DIM)),
        full((CAT_DIM, NEWS_DIM)), full((1, NEWS_DIM)),
        full((CAT_DIM, NEWS_DIM)), full((1, NEWS_DIM)),
        full((NEWS_DIM, QPAD)), full((1, QPAD)), full((1, QPAD)),
        full((1, 1)),
    ]
    args = [news_rows, cat_rows, auth_rows, wp, bp, g, bb, wc, bc, wa, ba,
            w1, b1, w2, b2]
    aliases = {}
    if out_buf is not None:
        in_specs.append(pl.BlockSpec(memory_space=pl.ANY))
        args.append(out_buf)
        aliases = {15: 0}
    return pl.pallas_call(
        body,
        grid=grid,
        in_specs=in_specs,
        out_specs=pl.BlockSpec((BLK, NEWS_DIM), lambda i: (i + blk_off, 0)),
        out_shape=jax.ShapeDtypeStruct((B, NEWS_DIM), jnp.float32),
        input_output_aliases=aliases,
    )(*args)


def kernel(x, news_table, W_pool, b_pool, ln_g, ln_b, cat_table, W_cat,
           b_cat, auth_table, W_auth, b_auth, aW1, ab1, aW2, ab2):
    idx0 = x[:, 0]
    idx1 = x[:, 1]
    idx2 = x[:, 2]

    wp = W_pool.T
    wc = W_cat.T
    wa = W_auth.T
    w1 = jnp.zeros((NEWS_DIM, QPAD), jnp.float32).at[:, :QDIM].set(aW1.T)
    b1 = jnp.zeros((1, QPAD), jnp.float32).at[0, :QDIM].set(ab1)
    w2 = jnp.zeros((1, QPAD), jnp.float32).at[0, :QDIM].set(aW2[0])
    row = lambda v: v.reshape(1, -1)

    out = None
    lo = 0
    for rows in SPLITS:
        news_rows, cat_rows, auth_rows = _make_sc_gather(rows)(
            lax.dynamic_slice_in_dim(idx0, lo, rows),
            lax.dynamic_slice_in_dim(idx1, lo, rows),
            lax.dynamic_slice_in_dim(idx2, lo, rows),
            news_table, cat_table, auth_table)
        out = _dense(news_rows, cat_rows, auth_rows, wp, row(b_pool),
                     row(ln_g), row(ln_b), wc, row(b_cat), wa,
                     row(b_auth), w1, b1, w2, ab2.reshape(1, 1),
                     out, lo // BLK)
        lo += rows
    return out
